# baseline (speedup 1.0000x reference)
import functools
import math

import jax
import jax.numpy as jnp
from jax import lax
from jax.experimental import pallas as pl
from jax.experimental.pallas import tpu as pltpu
from jax.experimental.pallas import tpu_sc as plsc

D = 64
SCALE = math.sqrt(D)
NC, NS, L = 2, 16, 16
NW = NC * NS
V = 1000000
B_ROWS = 4096
SEQ = 200
CH = 512
NCH_FULL = V // CH
REM0 = NCH_FULL * CH
REM = V - REM0
BPW = B_ROWS // NW

_SC_PARAMS = pltpu.CompilerParams(needs_layout_passes=False)


@functools.partial(
    pl.kernel,
    mesh=plsc.VectorSubcoreMesh(core_axis_name="c", subcore_axis_name="s"),
    compiler_params=_SC_PARAMS,
    out_type=jax.ShapeDtypeStruct((V // 2, 2 * D), jnp.float32),
    scratch_types=[
        pltpu.VMEM((D, CH), jnp.float32),
        pltpu.VMEM((CH // 2, 2 * D), jnp.float32),
        pltpu.VMEM((D, REM), jnp.float32),
    ],
)
def _transpose_table(tt_hbm, tail_hbm, out_hbm, in_v, out_v, tail_v):
    c = lax.axis_index("c")
    s = lax.axis_index("s")
    wid = s * NC + c
    lanes = lax.iota(jnp.int32, L)
    rows = [lanes + d0 * L for d0 in range(D // L)]

    def do_cols(src_v, ncols):
        def col(cc, carry):
            cols = jnp.full((L,), cc, jnp.int32)
            half = (cc & 1) * D
            for d0 in range(D // L):
                v = plsc.load_gather(src_v, [rows[d0], cols])
                out_v[lax.shift_right_logical(cc, 1),
                      pl.ds(half + d0 * L, L)] = v
            return carry

        lax.fori_loop(0, ncols, col, 0)

    def chunk(k, carry):
        ch = k * NW + wid

        @pl.when(ch < NCH_FULL)
        def _():
            c0 = pl.multiple_of(ch * CH, CH)
            pltpu.sync_copy(tt_hbm.at[:, pl.ds(c0, CH)], in_v)
            do_cols(in_v, CH)
            pltpu.sync_copy(out_v, out_hbm.at[pl.ds(pl.multiple_of(ch * (CH // 2), 8), CH // 2)])

        return carry

    lax.fori_loop(0, (NCH_FULL + NW - 1) // NW, chunk, 0)

    @pl.when(wid == NW - 1)
    def _():
        pltpu.sync_copy(tail_hbm, tail_v)
        do_cols(tail_v, REM)
        pltpu.sync_copy(out_v.at[pl.ds(0, REM // 2)],
                        out_hbm.at[pl.ds(REM0 // 2, REM // 2)])


@functools.partial(
    pl.kernel,
    mesh=plsc.VectorSubcoreMesh(core_axis_name="c", subcore_axis_name="s"),
    compiler_params=_SC_PARAMS,
    out_type=jax.ShapeDtypeStruct((SEQ, D, B_ROWS), jnp.float32),
    scratch_types=[
        pltpu.VMEM((SEQ, BPW), jnp.int32),
        pltpu.VMEM((BPW,), jnp.int32),
        pltpu.VMEM((BPW, 2 * D), jnp.float32),
        pltpu.VMEM((D, BPW), jnp.float32),
        pltpu.SemaphoreType.DMA,
    ],
)
def _gather_scaled(xt_hbm, table_hbm, out_hbm, idx_v, idx2_v, buf_v, slab_v,
                   sem):
    c = lax.axis_index("c")
    s = lax.axis_index("s")
    wid = s * NC + c
    b0 = pl.multiple_of(wid * BPW, BPW)
    pltpu.sync_copy(xt_hbm.at[:, pl.ds(b0, BPW)], idx_v)
    lanes = lax.iota(jnp.int32, L)
    brows = [lanes + bb * L for bb in range(BPW // L)]

    def step(t, carry):
        def halve(g, carry2):
            iv = idx_v[t, pl.ds(g * L, L)]
            idx2_v[pl.ds(g * L, L)] = lax.shift_right_logical(iv, 1)
            return carry2

        lax.fori_loop(0, BPW // L, halve, 0)
        pltpu.async_copy(table_hbm.at[idx2_v], buf_v, sem).wait()

        for bb in range(BPW // L):
            iv = idx_v[t, pl.ds(bb * L, L)]
            pbase = (iv & 1) << 6
            for d in range(D):
                v = plsc.load_gather(buf_v, [brows[bb], pbase + d])
                slab_v[d, pl.ds(bb * L, L)] = v * SCALE

        pltpu.sync_copy(slab_v, out_hbm.at[t, :, pl.ds(b0, BPW)])
        return carry

    lax.fori_loop(0, SEQ, step, 0)


def kernel(x, table):
    xt = x.T.astype(jnp.int32)
    tt = table.T
    tail = table[REM0:, :].T
    t64 = _transpose_table(tt, tail)
    ok = _gather_scaled(xt, t64)
    return ok.transpose(2, 0, 1)


# speedup vs baseline: 1.2627x; 1.2627x over previous
import functools
import math

import jax
import jax.numpy as jnp
from jax import lax
from jax.experimental import pallas as pl
from jax.experimental.pallas import tpu as pltpu
from jax.experimental.pallas import tpu_sc as plsc

D = 64
SCALE = math.sqrt(D)
NC, NS, L = 2, 16, 16
NW = NC * NS
V = 1000000
B_ROWS = 4096
SEQ = 200
CH = 384
NCH_FULL = V // CH
REM0 = NCH_FULL * CH
REM = V - REM0
NK_BASE = NCH_FULL // NW
NK_EXTRA = NCH_FULL - NK_BASE * NW
BPW = B_ROWS // NW

_SC_PARAMS = pltpu.CompilerParams(needs_layout_passes=False)


@functools.partial(
    pl.kernel,
    mesh=plsc.VectorSubcoreMesh(core_axis_name="c", subcore_axis_name="s"),
    compiler_params=_SC_PARAMS,
    out_type=jax.ShapeDtypeStruct((V // 2, 2 * D), jnp.float32),
    scratch_types=[
        pltpu.VMEM((D, CH), jnp.float32),
        pltpu.VMEM((D, CH), jnp.float32),
        pltpu.VMEM((CH // 2, 2 * D), jnp.float32),
        pltpu.VMEM((CH // 2, 2 * D), jnp.float32),
        pltpu.VMEM((D, REM), jnp.float32),
        pltpu.SemaphoreType.DMA,
        pltpu.SemaphoreType.DMA,
        pltpu.SemaphoreType.DMA,
        pltpu.SemaphoreType.DMA,
    ],
)
def _transpose_table(tt_hbm, tail_hbm, out_hbm,
                     in0, in1, out0, out1, tail_v, g0, g1, s0, s1):
    c = lax.axis_index("c")
    s = lax.axis_index("s")
    wid = s * NC + c
    base = wid * NK_BASE + jnp.minimum(wid, NK_EXTRA)
    nk = NK_BASE + (wid < NK_EXTRA).astype(jnp.int32)
    lanes = lax.iota(jnp.int32, L)
    ins, outs, gs, ss = (in0, in1), (out0, out1), (g0, g1), (s0, s1)

    def start_in(k, b):
        c0 = pl.multiple_of((base + k) * CH, 128)
        pltpu.async_copy(tt_hbm.at[:, pl.ds(c0, CH)], ins[b], gs[b])

    def start_out(k, b):
        r0 = pl.multiple_of((base + k) * (CH // 2), 8)
        pltpu.async_copy(outs[b], out_hbm.at[pl.ds(r0, CH // 2)], ss[b])

    def wait_in(b):
        pltpu.make_async_copy(tt_hbm.at[:, pl.ds(0, CH)], ins[b],
                              gs[b]).wait()

    def wait_out(b):
        pltpu.make_async_copy(outs[b], out_hbm.at[pl.ds(0, CH // 2)],
                              ss[b]).wait()

    def transpose(src_v, dst_v, ncols):
        def blk(q, carry):
            cl = lanes + q * L
            rows = lax.shift_right_logical(cl, 1)
            cb = (cl & 1) << 6
            for d in range(D):
                v = src_v[d, pl.ds(pl.multiple_of(q * L, L), L)]
                plsc.store_scatter(dst_v, [rows, cb + d], v)
            return carry

        lax.fori_loop(0, ncols // L, blk, 0)

    start_in(0, 0)
    start_in(1, 1)

    def pair(p, carry):
        for b in range(2):
            k = 2 * p + b
            wait_in(b)

            @pl.when(k >= 2)
            def _():
                wait_out(b)

            transpose(ins[b], outs[b], CH)
            start_out(k, b)

            @pl.when(k + 2 < NK_BASE)
            def _():
                start_in(k + 2, b)

        return carry

    lax.fori_loop(0, NK_BASE // 2, pair, 0)
    kl = NK_BASE - 1
    bl = kl % 2
    wait_in(bl)
    wait_out(bl)
    transpose(ins[bl], outs[bl], CH)
    start_out(kl, bl)
    wait_out(1 - bl)
    wait_out(bl)

    @pl.when(wid < NK_EXTRA)
    def _():
        pltpu.sync_copy(tt_hbm.at[:, pl.ds(
            pl.multiple_of((base + NK_BASE) * CH, 128), CH)], in0)
        transpose(in0, out0, CH)
        pltpu.sync_copy(out0, out_hbm.at[pl.ds(
            pl.multiple_of((base + NK_BASE) * (CH // 2), 8), CH // 2)])

    @pl.when(wid == NW - 1)
    def _():
        pltpu.sync_copy(tail_hbm, tail_v)

        def col(cc, carry):
            cols = jnp.full((L,), cc, jnp.int32)
            half = (cc & 1) * D
            for d0 in range(D // L):
                v = plsc.load_gather(tail_v, [lanes + d0 * L, cols])
                out0[lax.shift_right_logical(cc, 1),
                     pl.ds(half + d0 * L, L)] = v
            return carry

        lax.fori_loop(0, REM, col, 0)
        pltpu.sync_copy(out0.at[pl.ds(0, REM // 2)],
                        out_hbm.at[pl.ds(REM0 // 2, REM // 2)])


@functools.partial(
    pl.kernel,
    mesh=plsc.VectorSubcoreMesh(core_axis_name="c", subcore_axis_name="s"),
    compiler_params=_SC_PARAMS,
    out_type=jax.ShapeDtypeStruct((SEQ, D, B_ROWS), jnp.float32),
    scratch_types=[
        pltpu.VMEM((SEQ, BPW), jnp.int32),
        pltpu.VMEM((BPW,), jnp.int32),
        pltpu.VMEM((BPW,), jnp.int32),
        pltpu.VMEM((BPW, 2 * D), jnp.float32),
        pltpu.VMEM((BPW, 2 * D), jnp.float32),
        pltpu.VMEM((D, BPW), jnp.float32),
        pltpu.VMEM((D, BPW), jnp.float32),
        pltpu.SemaphoreType.DMA,
        pltpu.SemaphoreType.DMA,
        pltpu.SemaphoreType.DMA,
        pltpu.SemaphoreType.DMA,
    ],
)
def _gather_scaled(xt_hbm, table_hbm, out_hbm, idx_v, ix0, ix1,
                   buf0, buf1, sl0, sl1, g0, g1, s0, s1):
    c = lax.axis_index("c")
    s = lax.axis_index("s")
    wid = s * NC + c
    b0 = pl.multiple_of(wid * BPW, BPW)
    pltpu.sync_copy(xt_hbm.at[:, pl.ds(b0, BPW)], idx_v)
    lanes = lax.iota(jnp.int32, L)
    brows = [lanes + bb * L for bb in range(BPW // L)]
    ixs, bufs, slabs, gs, ss = (ix0, ix1), (buf0, buf1), (sl0, sl1), \
        (g0, g1), (s0, s1)

    def start_gather(t, b):
        def halve(g, carry):
            iv = idx_v[t, pl.ds(g * L, L)]
            ixs[b][pl.ds(g * L, L)] = lax.shift_right_logical(iv, 1)
            return carry

        lax.fori_loop(0, BPW // L, halve, 0)
        pltpu.async_copy(table_hbm.at[ixs[b]], bufs[b], gs[b])

    def wait_gather(b):
        pltpu.make_async_copy(table_hbm.at[pl.ds(0, BPW)], bufs[b],
                              gs[b]).wait()

    def start_out(t, b):
        pltpu.async_copy(slabs[b], out_hbm.at[t, :, pl.ds(b0, BPW)], ss[b])

    def wait_out(b):
        pltpu.make_async_copy(slabs[b], out_hbm.at[0, :, pl.ds(b0, BPW)],
                              ss[b]).wait()

    start_gather(0, 0)
    start_gather(1, 1)

    def pair(p, carry):
        for b in range(2):
            t = 2 * p + b
            wait_gather(b)

            @pl.when(t >= 2)
            def _():
                wait_out(b)

            for bb in range(BPW // L):
                iv = idx_v[t, pl.ds(bb * L, L)]
                pbase = (iv & 1) << 6
                for d in range(D):
                    v = plsc.load_gather(bufs[b], [brows[bb], pbase + d])
                    slabs[b][d, pl.ds(bb * L, L)] = v * SCALE

            start_out(t, b)

            @pl.when(t + 2 < SEQ)
            def _():
                start_gather(t + 2, b)

        return carry

    lax.fori_loop(0, SEQ // 2, pair, 0)
    wait_out(0)
    wait_out(1)


def kernel(x, table):
    xt = x.T.astype(jnp.int32)
    tt = table.T
    tail = table[REM0:, :].T
    t64 = _transpose_table(tt, tail)
    ok = _gather_scaled(xt, t64)
    return ok.transpose(2, 0, 1)


# speedup vs baseline: 2.0287x; 1.6067x over previous
import functools
import math

import jax
import jax.numpy as jnp
from jax import lax
from jax.experimental import pallas as pl
from jax.experimental.pallas import tpu as pltpu
from jax.experimental.pallas import tpu_sc as plsc

D = 64
SCALE = math.sqrt(D)
NC, NS, L = 2, 16, 16
NW = NC * NS
V = 1000000
B_ROWS = 4096
SEQ = 200
CH = 384
NCH_FULL = V // CH
REM0 = NCH_FULL * CH
REM = V - REM0
NK_BASE = NCH_FULL // NW
NK_EXTRA = NCH_FULL - NK_BASE * NW
BPW = B_ROWS // NW

_SC_PARAMS = pltpu.CompilerParams(needs_layout_passes=False)


@functools.partial(
    pl.kernel,
    mesh=plsc.VectorSubcoreMesh(core_axis_name="c", subcore_axis_name="s"),
    compiler_params=_SC_PARAMS,
    out_type=jax.ShapeDtypeStruct((V // 2, 2 * D), jnp.float32),
    scratch_types=[
        pltpu.VMEM((D, CH), jnp.float32),
        pltpu.VMEM((D, CH), jnp.float32),
        pltpu.VMEM((CH // 2, 2 * D), jnp.float32),
        pltpu.VMEM((CH // 2, 2 * D), jnp.float32),
        pltpu.VMEM((D, REM), jnp.float32),
        pltpu.SemaphoreType.DMA,
        pltpu.SemaphoreType.DMA,
        pltpu.SemaphoreType.DMA,
        pltpu.SemaphoreType.DMA,
    ],
)
def _transpose_table(tt_hbm, tail_hbm, out_hbm,
                     in0, in1, out0, out1, tail_v, g0, g1, s0, s1):
    c = lax.axis_index("c")
    s = lax.axis_index("s")
    wid = s * NC + c
    base = wid * NK_BASE + jnp.minimum(wid, NK_EXTRA)
    nk = NK_BASE + (wid < NK_EXTRA).astype(jnp.int32)
    lanes = lax.iota(jnp.int32, L)
    ins, outs, gs, ss = (in0, in1), (out0, out1), (g0, g1), (s0, s1)

    def start_in(k, b):
        c0 = pl.multiple_of((base + k) * CH, 128)
        pltpu.async_copy(tt_hbm.at[:, pl.ds(c0, CH)], ins[b], gs[b])

    def start_out(k, b):
        r0 = pl.multiple_of((base + k) * (CH // 2), 8)
        pltpu.async_copy(outs[b], out_hbm.at[pl.ds(r0, CH // 2)], ss[b])

    def wait_in(b):
        pltpu.make_async_copy(tt_hbm.at[:, pl.ds(0, CH)], ins[b],
                              gs[b]).wait()

    def wait_out(b):
        pltpu.make_async_copy(outs[b], out_hbm.at[pl.ds(0, CH // 2)],
                              ss[b]).wait()

    def transpose(src_v, dst_v, ncols):
        def blk(q, carry):
            cl = lanes + q * L
            rows = lax.shift_right_logical(cl, 1)
            cb = (cl & 1) << 6
            for d0 in range(0, D, L):
                vs = [src_v[d0 + i, pl.ds(pl.multiple_of(q * L, L), L)]
                      for i in range(L)]
                for i in range(L):
                    plsc.store_scatter(dst_v, [rows, cb + (d0 + i)], vs[i])
            return carry

        lax.fori_loop(0, ncols // L, blk, 0)

    start_in(0, 0)
    start_in(1, 1)

    def pair(p, carry):
        for b in range(2):
            k = 2 * p + b
            wait_in(b)

            @pl.when(k >= 2)
            def _():
                wait_out(b)

            transpose(ins[b], outs[b], CH)
            start_out(k, b)

            @pl.when(k + 2 < NK_BASE)
            def _():
                start_in(k + 2, b)

        return carry

    lax.fori_loop(0, NK_BASE // 2, pair, 0)
    kl = NK_BASE - 1
    bl = kl % 2
    wait_in(bl)
    wait_out(bl)
    transpose(ins[bl], outs[bl], CH)
    start_out(kl, bl)
    wait_out(1 - bl)
    wait_out(bl)

    @pl.when(wid < NK_EXTRA)
    def _():
        pltpu.sync_copy(tt_hbm.at[:, pl.ds(
            pl.multiple_of((base + NK_BASE) * CH, 128), CH)], in0)
        transpose(in0, out0, CH)
        pltpu.sync_copy(out0, out_hbm.at[pl.ds(
            pl.multiple_of((base + NK_BASE) * (CH // 2), 8), CH // 2)])

    @pl.when(wid == NW - 1)
    def _():
        pltpu.sync_copy(tail_hbm, tail_v)

        def col(cc, carry):
            cols = jnp.full((L,), cc, jnp.int32)
            half = (cc & 1) * D
            for d0 in range(D // L):
                v = plsc.load_gather(tail_v, [lanes + d0 * L, cols])
                out0[lax.shift_right_logical(cc, 1),
                     pl.ds(half + d0 * L, L)] = v
            return carry

        lax.fori_loop(0, REM, col, 0)
        pltpu.sync_copy(out0.at[pl.ds(0, REM // 2)],
                        out_hbm.at[pl.ds(REM0 // 2, REM // 2)])


@functools.partial(
    pl.kernel,
    mesh=plsc.VectorSubcoreMesh(core_axis_name="c", subcore_axis_name="s"),
    compiler_params=_SC_PARAMS,
    out_type=jax.ShapeDtypeStruct((SEQ, D, B_ROWS), jnp.float32),
    scratch_types=[
        pltpu.VMEM((SEQ, BPW), jnp.int32),
        pltpu.VMEM((BPW,), jnp.int32),
        pltpu.VMEM((BPW,), jnp.int32),
        pltpu.VMEM((BPW, 2 * D), jnp.float32),
        pltpu.VMEM((BPW, 2 * D), jnp.float32),
        pltpu.VMEM((D, BPW), jnp.float32),
        pltpu.VMEM((D, BPW), jnp.float32),
        pltpu.SemaphoreType.DMA,
        pltpu.SemaphoreType.DMA,
        pltpu.SemaphoreType.DMA,
        pltpu.SemaphoreType.DMA,
    ],
)
def _gather_scaled(xt_hbm, table_hbm, out_hbm, idx_v, ix0, ix1,
                   buf0, buf1, sl0, sl1, g0, g1, s0, s1):
    c = lax.axis_index("c")
    s = lax.axis_index("s")
    wid = s * NC + c
    b0 = pl.multiple_of(wid * BPW, BPW)
    pltpu.sync_copy(xt_hbm.at[:, pl.ds(b0, BPW)], idx_v)
    lanes = lax.iota(jnp.int32, L)
    brows = [lanes + bb * L for bb in range(BPW // L)]
    ixs, bufs, slabs, gs, ss = (ix0, ix1), (buf0, buf1), (sl0, sl1), \
        (g0, g1), (s0, s1)

    def start_gather(t, b):
        def halve(g, carry):
            iv = idx_v[t, pl.ds(g * L, L)]
            ixs[b][pl.ds(g * L, L)] = lax.shift_right_logical(iv, 1)
            return carry

        lax.fori_loop(0, BPW // L, halve, 0)
        pltpu.async_copy(table_hbm.at[ixs[b]], bufs[b], gs[b])

    def wait_gather(b):
        pltpu.make_async_copy(table_hbm.at[pl.ds(0, BPW)], bufs[b],
                              gs[b]).wait()

    def start_out(t, b):
        pltpu.async_copy(slabs[b], out_hbm.at[t, :, pl.ds(b0, BPW)], ss[b])

    def wait_out(b):
        pltpu.make_async_copy(slabs[b], out_hbm.at[0, :, pl.ds(b0, BPW)],
                              ss[b]).wait()

    start_gather(0, 0)
    start_gather(1, 1)

    def pair(p, carry):
        for b in range(2):
            t = 2 * p + b
            wait_gather(b)

            @pl.when(t >= 2)
            def _():
                wait_out(b)

            for bb in range(BPW // L):
                iv = idx_v[t, pl.ds(bb * L, L)]
                pbase = (iv & 1) << 6
                for d0 in range(0, D, L):
                    vs = [plsc.load_gather(bufs[b],
                                           [brows[bb], pbase + (d0 + i)])
                          for i in range(L)]
                    for i in range(L):
                        slabs[b][d0 + i, pl.ds(bb * L, L)] = vs[i] * SCALE

            start_out(t, b)

            @pl.when(t + 2 < SEQ)
            def _():
                start_gather(t + 2, b)

        return carry

    lax.fori_loop(0, SEQ // 2, pair, 0)
    wait_out(0)
    wait_out(1)


def kernel(x, table):
    xt = x.T.astype(jnp.int32)
    tt = table.T
    tail = table[REM0:, :].T
    t64 = _transpose_table(tt, tail)
    ok = _gather_scaled(xt, t64)
    return ok.transpose(2, 0, 1)


# speedup vs baseline: 2.5588x; 1.2613x over previous
import functools
import math

import jax
import jax.numpy as jnp
from jax import lax
from jax.experimental import pallas as pl
from jax.experimental.pallas import tpu as pltpu
from jax.experimental.pallas import tpu_sc as plsc

D = 64
SCALE = math.sqrt(D)
NC, NS, L = 2, 16, 16
NW = NC * NS
V = 1000000
B_ROWS = 4096
SEQ = 200
CH = 384
NCH_FULL = V // CH
REM0 = NCH_FULL * CH
REM = V - REM0
NK_BASE = NCH_FULL // NW
NK_EXTRA = NCH_FULL - NK_BASE * NW
BPW = B_ROWS // NW

_SC_PARAMS = pltpu.CompilerParams(needs_layout_passes=False)


@functools.partial(
    pl.kernel,
    mesh=plsc.VectorSubcoreMesh(core_axis_name="c", subcore_axis_name="s"),
    compiler_params=_SC_PARAMS,
    out_type=jax.ShapeDtypeStruct((V // 2, 2 * D), jnp.float32),
    scratch_types=[
        pltpu.VMEM((D, CH), jnp.float32),
        pltpu.VMEM((D, CH), jnp.float32),
        pltpu.VMEM((CH // 2, 2 * D), jnp.float32),
        pltpu.VMEM((CH // 2, 2 * D), jnp.float32),
        pltpu.VMEM((D, REM), jnp.float32),
        pltpu.SemaphoreType.DMA,
        pltpu.SemaphoreType.DMA,
        pltpu.SemaphoreType.DMA,
        pltpu.SemaphoreType.DMA,
    ],
)
def _transpose_table(tt_hbm, tail_hbm, out_hbm,
                     in0, in1, out0, out1, tail_v, g0, g1, s0, s1):
    c = lax.axis_index("c")
    s = lax.axis_index("s")
    wid = s * NC + c
    base = wid * NK_BASE + jnp.minimum(wid, NK_EXTRA)
    nk = NK_BASE + (wid < NK_EXTRA).astype(jnp.int32)
    lanes = lax.iota(jnp.int32, L)
    ins, outs, gs, ss = (in0, in1), (out0, out1), (g0, g1), (s0, s1)

    def start_in(k, b):
        c0 = pl.multiple_of((base + k) * CH, 128)
        pltpu.async_copy(tt_hbm.at[:, pl.ds(c0, CH)], ins[b], gs[b])

    def start_out(k, b):
        r0 = pl.multiple_of((base + k) * (CH // 2), 8)
        pltpu.async_copy(outs[b], out_hbm.at[pl.ds(r0, CH // 2)], ss[b])

    def wait_in(b):
        pltpu.make_async_copy(tt_hbm.at[:, pl.ds(0, CH)], ins[b],
                              gs[b]).wait()

    def wait_out(b):
        pltpu.make_async_copy(outs[b], out_hbm.at[pl.ds(0, CH // 2)],
                              ss[b]).wait()

    def transpose(src_v, dst_v, ncols):
        def blk(q, carry):
            cl = lanes + q * L
            rows = lax.shift_right_logical(cl, 1)
            cb = (cl & 1) << 6
            for d0 in range(0, D, L):
                vs = [src_v[d0 + i, pl.ds(pl.multiple_of(q * L, L), L)]
                      for i in range(L)]
                for i in range(L):
                    plsc.store_scatter(dst_v, [rows, cb + (d0 + i)], vs[i])
            return carry

        lax.fori_loop(0, ncols // L, blk, 0)

    start_in(0, 0)
    start_in(1, 1)

    def pair(p, carry):
        for b in range(2):
            k = 2 * p + b
            wait_in(b)

            @pl.when(k >= 2)
            def _():
                wait_out(b)

            transpose(ins[b], outs[b], CH)
            start_out(k, b)

            @pl.when(k + 2 < NK_BASE)
            def _():
                start_in(k + 2, b)

        return carry

    lax.fori_loop(0, NK_BASE // 2, pair, 0)
    kl = NK_BASE - 1
    bl = kl % 2
    wait_in(bl)
    wait_out(bl)
    transpose(ins[bl], outs[bl], CH)
    start_out(kl, bl)
    wait_out(1 - bl)
    wait_out(bl)

    @pl.when(wid < NK_EXTRA)
    def _():
        pltpu.sync_copy(tt_hbm.at[:, pl.ds(
            pl.multiple_of((base + NK_BASE) * CH, 128), CH)], in0)
        transpose(in0, out0, CH)
        pltpu.sync_copy(out0, out_hbm.at[pl.ds(
            pl.multiple_of((base + NK_BASE) * (CH // 2), 8), CH // 2)])

    @pl.when(wid == NW - 1)
    def _():
        pltpu.sync_copy(tail_hbm, tail_v)

        def col(cc, carry):
            cols = jnp.full((L,), cc, jnp.int32)
            half = (cc & 1) * D
            for d0 in range(D // L):
                v = plsc.load_gather(tail_v, [lanes + d0 * L, cols])
                out0[lax.shift_right_logical(cc, 1),
                     pl.ds(half + d0 * L, L)] = v
            return carry

        lax.fori_loop(0, REM, col, 0)
        pltpu.sync_copy(out0.at[pl.ds(0, REM // 2)],
                        out_hbm.at[pl.ds(REM0 // 2, REM // 2)])


@functools.partial(
    pl.kernel,
    mesh=plsc.VectorSubcoreMesh(core_axis_name="c", subcore_axis_name="s"),
    compiler_params=_SC_PARAMS,
    out_type=jax.ShapeDtypeStruct((SEQ, D, B_ROWS), jnp.float32),
    scratch_types=[
        pltpu.VMEM((SEQ, BPW), jnp.int32),
        pltpu.VMEM((BPW,), jnp.int32),
        pltpu.VMEM((BPW,), jnp.int32),
        pltpu.VMEM((BPW, 2 * D), jnp.float32),
        pltpu.VMEM((BPW, 2 * D), jnp.float32),
        pltpu.VMEM((D, BPW), jnp.float32),
        pltpu.VMEM((D, BPW), jnp.float32),
        pltpu.SemaphoreType.DMA,
        pltpu.SemaphoreType.DMA,
        pltpu.SemaphoreType.DMA,
        pltpu.SemaphoreType.DMA,
    ],
)
def _gather_scaled(xt_hbm, table_hbm, out_hbm, idx_v, ix0, ix1,
                   buf0, buf1, sl0, sl1, g0, g1, s0, s1):
    c = lax.axis_index("c")
    s = lax.axis_index("s")
    wid = s * NC + c
    b0 = pl.multiple_of(wid * BPW, BPW)
    pltpu.sync_copy(xt_hbm.at[:, pl.ds(b0, BPW)], idx_v)
    lanes = lax.iota(jnp.int32, L)
    brows = [lanes + bb * L for bb in range(BPW // L)]
    ixs, bufs, slabs, gs, ss = (ix0, ix1), (buf0, buf1), (sl0, sl1), \
        (g0, g1), (s0, s1)

    def start_gather(t, b):
        def halve(g, carry):
            iv = idx_v[t, pl.ds(g * L, L)]
            ixs[b][pl.ds(g * L, L)] = lax.shift_right_logical(iv, 1)
            return carry

        lax.fori_loop(0, BPW // L, halve, 0)
        pltpu.async_copy(table_hbm.at[ixs[b]], bufs[b], gs[b])

    def wait_gather(b):
        pltpu.make_async_copy(table_hbm.at[pl.ds(0, BPW)], bufs[b],
                              gs[b]).wait()

    def start_out(t, b):
        pltpu.async_copy(slabs[b], out_hbm.at[t, :, pl.ds(b0, BPW)], ss[b])

    def wait_out(b):
        pltpu.make_async_copy(slabs[b], out_hbm.at[0, :, pl.ds(b0, BPW)],
                              ss[b]).wait()

    start_gather(0, 0)
    start_gather(1, 1)

    def pair(p, carry):
        for b in range(2):
            t = 2 * p + b
            wait_gather(b)

            @pl.when(t >= 2)
            def _():
                wait_out(b)

            for bb in range(BPW // L):
                iv = idx_v[t, pl.ds(bb * L, L)]
                pbase = (iv & 1) << 6
                for d0 in range(0, D, L):
                    vs = [plsc.load_gather(bufs[b],
                                           [brows[bb], pbase + (d0 + i)])
                          for i in range(L)]
                    for i in range(L):
                        slabs[b][d0 + i, pl.ds(bb * L, L)] = vs[i] * SCALE

            start_out(t, b)

            @pl.when(t + 2 < SEQ)
            def _():
                start_gather(t + 2, b)

        return carry

    lax.fori_loop(0, SEQ // 2, pair, 0)
    wait_out(0)
    wait_out(1)


def kernel(x, table):
    xt = x.T.astype(jnp.int32)
    t64 = table.reshape(V // 2, 2 * D)
    ok = _gather_scaled(xt, t64)
    return ok.transpose(2, 0, 1)
